# baseline (speedup 1.0000x reference)
import functools

import jax
import jax.numpy as jnp
from jax import lax
from jax.experimental import pallas as pl
from jax.experimental.pallas import tpu as pltpu

_V = 10000
_D = 10000
_B = 2048


def _tc_body(idx_smem, sims_hbm, out_hbm, sem):
    def issue(i, carry):
        pltpu.make_async_copy(
            sims_hbm.at[pl.ds(idx_smem[i], 1)],
            out_hbm.at[pl.ds(i, 1)],
            sem,
        ).start()
        return carry

    def drain(i, carry):
        pltpu.make_async_copy(
            sims_hbm.at[pl.ds(0, 1)],
            out_hbm.at[pl.ds(0, 1)],
            sem,
        ).wait()
        return carry

    lax.fori_loop(0, _B, issue, 0)
    lax.fori_loop(0, _B, drain, 0)


def kernel(x, sims):
    idx = x.reshape(-1).astype(jnp.int32)
    out = pl.pallas_call(
        _tc_body,
        grid_spec=pltpu.PrefetchScalarGridSpec(
            num_scalar_prefetch=1,
            grid=(1,),
            in_specs=[pl.BlockSpec(memory_space=pltpu.MemorySpace.HBM)],
            out_specs=pl.BlockSpec(memory_space=pltpu.MemorySpace.HBM),
            scratch_shapes=[pltpu.SemaphoreType.DMA],
        ),
        out_shape=jax.ShapeDtypeStruct((_B, _D), jnp.float32),
    )(idx, sims)
    return out.reshape(x.shape[0], x.shape[1], _V)


# speedup vs baseline: 29.0306x; 29.0306x over previous
import functools

import jax
import jax.numpy as jnp
from jax import lax
from jax.experimental import pallas as pl
from jax.experimental.pallas import tpu as pltpu
from jax.experimental.pallas import tpu_sc as plsc

_V = 10000
_D = 10000
_DM = 9984
_TW = 128
_TR = _D - _DM
_B = 2048
_NC = 2
_NS = 16
_NW = _NC * _NS
_BPW = _B // _NW
_K = 8
_NCHUNK = _BPW // _K


@functools.partial(
    pl.kernel,
    out_type=jax.ShapeDtypeStruct((_B, _D), jnp.float32),
    mesh=plsc.VectorSubcoreMesh(core_axis_name="c", subcore_axis_name="s"),
    scratch_types=[
        pltpu.VMEM((_BPW,), jnp.int32),
        pltpu.VMEM((_K, _D), jnp.float32),
        pltpu.VMEM((_K, _TW), jnp.float32),
        pltpu.SemaphoreType.DMA,
        pltpu.SemaphoreType.DMA,
    ],
)
def _gather_rows(sims_hbm, tail_hbm, idx_hbm, out_hbm, idx_v, rows_v, tail_v,
                 gsem, tsem):
    wid = lax.axis_index("s") * _NC + lax.axis_index("c")
    base = wid * _BPW
    pltpu.sync_copy(idx_hbm.at[pl.ds(base, _BPW)], idx_v)
    for c in range(_NCHUNK):
        idx_c = idx_v.at[pl.ds(c * _K, _K)]
        gh = pltpu.async_copy(
            sims_hbm.at[idx_c, pl.ds(0, _DM)], rows_v.at[:, pl.ds(0, _DM)],
            gsem,
        )
        th = pltpu.async_copy(tail_hbm.at[idx_c], tail_v, tsem)
        gh.wait()
        th.wait()
        for r in range(_K):
            rows_v[r, pl.ds(_DM, _TR)] = tail_v[r, pl.ds(_TW - _TR, _TR)]
        pltpu.sync_copy(rows_v, out_hbm.at[pl.ds(base + c * _K, _K)])


def kernel(x, sims):
    idx = x.reshape(-1).astype(jnp.int32)
    tail = lax.slice(sims, (0, _D - _TW), (_V, _D))
    out = _gather_rows(sims, tail, idx)
    return out.reshape(x.shape[0], x.shape[1], _V)


# speedup vs baseline: 29.2242x; 1.0067x over previous
import functools

import jax
import jax.numpy as jnp
from jax import lax
from jax.experimental import pallas as pl
from jax.experimental.pallas import tpu as pltpu
from jax.experimental.pallas import tpu_sc as plsc

_V = 10000
_D = 10000
_DM = 9984
_TW = 128
_TR = _D - _DM
_B = 2048
_NC = 2
_NS = 16
_NW = _NC * _NS
_BPW = _B // _NW
_K = 8
_NCHUNK = _BPW // _K


@functools.partial(
    pl.kernel,
    out_type=jax.ShapeDtypeStruct((_B, _D), jnp.float32),
    mesh=plsc.VectorSubcoreMesh(core_axis_name="c", subcore_axis_name="s"),
    scratch_types=[
        pltpu.VMEM((_BPW,), jnp.int32),
        pltpu.VMEM((_K, _D), jnp.float32),
        pltpu.VMEM((_BPW, _TW), jnp.float32),
        pltpu.SemaphoreType.DMA,
        pltpu.SemaphoreType.DMA,
    ],
)
def _gather_rows(sims_hbm, tail_hbm, idx_hbm, out_hbm, idx_v, rows_v, tail_v,
                 gsem, tsem):
    wid = lax.axis_index("s") * _NC + lax.axis_index("c")
    base = wid * _BPW
    pltpu.sync_copy(idx_hbm.at[pl.ds(base, _BPW)], idx_v)
    th = pltpu.async_copy(tail_hbm.at[idx_v], tail_v, tsem)
    for c in range(_NCHUNK):
        idx_c = idx_v.at[pl.ds(c * _K, _K)]
        gh = pltpu.async_copy(
            sims_hbm.at[idx_c, pl.ds(0, _DM)], rows_v.at[:, pl.ds(0, _DM)],
            gsem,
        )
        gh.wait()
        if c == 0:
            th.wait()
        for r in range(_K):
            rows_v[r, pl.ds(_DM, _TR)] = tail_v[c * _K + r,
                                                pl.ds(_TW - _TR, _TR)]
        pltpu.sync_copy(rows_v, out_hbm.at[pl.ds(base + c * _K, _K)])


def kernel(x, sims):
    idx = x.reshape(-1).astype(jnp.int32)
    tail = lax.slice(sims, (0, _D - _TW), (_V, _D))
    out = _gather_rows(sims, tail, idx)
    return out.reshape(x.shape[0], x.shape[1], _V)
